# baseline (speedup 1.0000x reference)
import functools

import numpy as np
import jax
import jax.numpy as jnp
from jax import lax
from jax.experimental import pallas as pl
from jax.experimental.pallas import tpu as pltpu
from jax.experimental.pallas import tpu_sc as plsc

_VOLUME = 8
_C = 16
_ROW = _VOLUME * _C
_NC = 2
_NS = 16
_NW = _NC * _NS
_CHUNK_ROWS = 128
_CHUNK_W = _CHUNK_ROWS * _ROW


def _perm_indices():
    lane = np.arange(16)
    srcs = np.zeros((_VOLUME, 16), np.int32)
    dsts = np.zeros((_VOLUME, 16), np.int32)
    for k in range(_VOLUME):
        i = (lane // 2 + k) % _VOLUME
        srcs[k] = i * _C + lane
        dsts[k] = lane * _VOLUME + i
    return np.concatenate([srcs, dsts], axis=0)


def _sc_body(x_hbm, idx_hbm, o_hbm, in_v, out_v, idx_v):
    wid = lax.axis_index("s") * _NC + lax.axis_index("c")
    n_rows = x_hbm.shape[0] // _ROW
    rows_per_w = n_rows // _NW
    n_chunks = rows_per_w // _CHUNK_ROWS
    base_w = wid * rows_per_w * _ROW

    pltpu.sync_copy(idx_hbm, idx_v)
    srcs = [idx_v[k, :] for k in range(_VOLUME)]
    dsts = [idx_v[_VOLUME + k, :] for k in range(_VOLUME)]

    def chunk_body(g, _):
        woff = base_w + g * _CHUNK_W
        pltpu.sync_copy(x_hbm.at[pl.ds(woff, _CHUNK_W)], in_v)

        def row_body(n, carry):
            svecs, dvecs = carry
            for k in range(_VOLUME):
                vals = plsc.load_gather(in_v, [svecs[k]])
                plsc.store_scatter(out_v, [dvecs[k]], vals)
            svecs = tuple(s + _ROW for s in svecs)
            dvecs = tuple(d + _ROW for d in dvecs)
            return (svecs, dvecs)

        lax.fori_loop(0, _CHUNK_ROWS, row_body, (tuple(srcs), tuple(dsts)),
                      unroll=2)
        pltpu.sync_copy(out_v, o_hbm.at[pl.ds(woff, _CHUNK_W)])
        return 0

    lax.fori_loop(0, n_chunks, chunk_body, 0)


def kernel(features, original_indices):
    n_rows = features.shape[0] // _VOLUME
    x = features.reshape(-1)
    mesh = plsc.VectorSubcoreMesh(core_axis_name="c", subcore_axis_name="s")
    out = pl.kernel(
        _sc_body,
        out_type=jax.ShapeDtypeStruct((n_rows * _ROW,), jnp.float32),
        mesh=mesh,
        compiler_params=pltpu.CompilerParams(needs_layout_passes=False),
        scratch_types=[
            pltpu.VMEM((_CHUNK_W,), jnp.float32),
            pltpu.VMEM((_CHUNK_W,), jnp.float32),
            pltpu.VMEM((16, 16), jnp.int32),
        ],
    )(x, jnp.asarray(_perm_indices()))
    return out.reshape(n_rows, _ROW), original_indices


# speedup vs baseline: 1.0049x; 1.0049x over previous
import numpy as np
import jax
import jax.numpy as jnp
from jax import lax
from jax.experimental import pallas as pl
from jax.experimental.pallas import tpu as pltpu
from jax.experimental.pallas import tpu_sc as plsc

_VOLUME = 8
_C = 16
_ROW = _VOLUME * _C
_NC = 2
_NS = 16
_NW = _NC * _NS
_CHUNK_ROWS = 128


def _sc_body(x_hbm, o_hbm, in_v, out_v):
    wid = lax.axis_index("s") * _NC + lax.axis_index("c")
    n_rows = o_hbm.shape[0]
    rows_per_w = n_rows // _NW
    n_chunks = rows_per_w // _CHUNK_ROWS
    base_row = wid * rows_per_w

    lane = lax.iota(jnp.int32, 16)
    srcs = []
    dsts = []
    for k in range(_VOLUME):
        i = ((lane >> 1) + k) & (_VOLUME - 1)
        srcs.append(i * _C + lane)
        dsts.append(lane * _VOLUME + i)
    zero16 = lane & 0

    def chunk_body(g, _):
        row0 = base_row + g * _CHUNK_ROWS
        pltpu.sync_copy(x_hbm.at[pl.ds(row0, _CHUNK_ROWS), :], in_v)

        def row_body(n, nvec):
            for k in range(_VOLUME):
                vals = plsc.load_gather(in_v, [nvec, srcs[k]])
                plsc.store_scatter(out_v, [nvec, dsts[k]], vals)
            return nvec + 1

        lax.fori_loop(0, _CHUNK_ROWS, row_body, zero16, unroll=2)
        pltpu.sync_copy(out_v, o_hbm.at[pl.ds(row0, _CHUNK_ROWS), :])
        return 0

    lax.fori_loop(0, n_chunks, chunk_body, 0)


def kernel(features, original_indices):
    n_rows = features.shape[0] // _VOLUME
    x = features.reshape(n_rows, _ROW)
    mesh = plsc.VectorSubcoreMesh(core_axis_name="c", subcore_axis_name="s")
    out = pl.kernel(
        _sc_body,
        out_type=jax.ShapeDtypeStruct((n_rows, _ROW), jnp.float32),
        mesh=mesh,
        compiler_params=pltpu.CompilerParams(needs_layout_passes=False),
        scratch_types=[
            pltpu.VMEM((_CHUNK_ROWS, _ROW), jnp.float32),
            pltpu.VMEM((_CHUNK_ROWS, _ROW), jnp.float32),
        ],
    )(x)
    return out, original_indices
